# Initial kernel scaffold; baseline (speedup 1.0000x reference)
import jax
import jax.numpy as jnp
from jax import lax
from jax.experimental import pallas as pl
from jax.experimental.pallas import tpu as pltpu
from jax.experimental.pallas import tpu_sc as plsc

N = 16384
C = 256
BATCH = 16

NUM_CORES = 2
NUM_SUBCORES = 16
NW = NUM_CORES * NUM_SUBCORES
ROWS = N // NW
CHUNK = 128
NCHUNK = ROWS // CHUNK
LANES = 16
CVEC = C // LANES


def _sc_body(patches_hbm, ids_hbm, parts_hbm, ids_v, buf0, buf1, acc_v,
             sem0, sem1):
    wid = lax.axis_index("s") * NUM_CORES + lax.axis_index("c")
    base = wid * ROWS

    pltpu.sync_copy(ids_hbm.at[pl.ds(base, ROWS)], ids_v)

    neg = jnp.full((LANES,), -jnp.inf, dtype=jnp.float32)
    for s in range(BATCH):
        for c in range(CVEC):
            acc_v[s, pl.ds(c * LANES, LANES)] = neg

    cnt = [jnp.zeros((LANES,), jnp.int32) for _ in range(BATCH - 1)]
    for j in range(ROWS // LANES):
        v = ids_v[pl.ds(j * LANES, LANES)]
        for s in range(1, BATCH):
            cnt[s - 1] = cnt[s - 1] + (v < s).astype(jnp.int32)
    bounds = [jnp.int32(0)]
    for s in range(1, BATCH):
        bounds.append(jnp.sum(cnt[s - 1]))
    bounds.append(jnp.int32(ROWS))

    bufs = [buf0, buf1]
    sems = [sem0, sem1]
    copies = [None, None]
    copies[0] = pltpu.async_copy(
        patches_hbm.at[pl.ds(base, CHUNK)], bufs[0], sems[0])
    for k in range(NCHUNK):
        cur = k % 2
        if k + 1 < NCHUNK:
            copies[(k + 1) % 2] = pltpu.async_copy(
                patches_hbm.at[pl.ds(base + (k + 1) * CHUNK, CHUNK)],
                bufs[(k + 1) % 2], sems[(k + 1) % 2])
        copies[cur].wait()
        buf = bufs[cur]
        for s in range(BATCH):
            lo = jnp.clip(bounds[s] - k * CHUNK, 0, CHUNK)
            hi = jnp.clip(bounds[s + 1] - k * CHUNK, 0, CHUNK)

            @pl.when(hi > lo)
            def _run(s=s, lo=lo, hi=hi, buf=buf):
                def body(r, carry):
                    return tuple(
                        jnp.maximum(carry[c], buf[r, pl.ds(c * LANES, LANES)])
                        for c in range(CVEC))
                init = tuple(
                    acc_v[s, pl.ds(c * LANES, LANES)] for c in range(CVEC))
                res = lax.fori_loop(lo, hi, body, init)
                for c in range(CVEC):
                    acc_v[s, pl.ds(c * LANES, LANES)] = res[c]

    pltpu.sync_copy(acc_v, parts_hbm.at[wid])


_sc_partials = pl.kernel(
    _sc_body,
    out_type=jax.ShapeDtypeStruct((NW, BATCH, C), jnp.float32),
    mesh=plsc.VectorSubcoreMesh(core_axis_name="c", subcore_axis_name="s"),
    scratch_types=[
        pltpu.VMEM((ROWS,), jnp.int32),
        pltpu.VMEM((CHUNK, C), jnp.float32),
        pltpu.VMEM((CHUNK, C), jnp.float32),
        pltpu.VMEM((BATCH, C), jnp.float32),
        pltpu.SemaphoreType.DMA,
        pltpu.SemaphoreType.DMA,
    ],
)


def _tc_merge_body(parts_ref, out_ref):
    out_ref[:] = jnp.max(parts_ref[:], axis=0)


def kernel(patches, patch_ids):
    parts = _sc_partials(patches, patch_ids)
    out = pl.pallas_call(
        _tc_merge_body,
        out_shape=jax.ShapeDtypeStruct((BATCH, C), jnp.float32),
    )(parts)
    return out


# speedup vs baseline: 2.2647x; 2.2647x over previous
import jax
import jax.numpy as jnp
from jax import lax
from jax.experimental import pallas as pl
from jax.experimental.pallas import tpu as pltpu
from jax.experimental.pallas import tpu_sc as plsc

N = 16384
C = 256
BATCH = 16

NUM_CORES = 2
NUM_SUBCORES = 16
NW = NUM_CORES * NUM_SUBCORES
ROWS = N // NW
CHUNK = 128
NCHUNK = ROWS // CHUNK
LANES = 16
CVEC = C // LANES


def _sc_body(patches_hbm, ids_hbm, parts_hbm, ids_v, buf0, buf1, acc_v,
             sem0, sem1):
    wid = lax.axis_index("s") * NUM_CORES + lax.axis_index("c")
    base = wid * ROWS

    pltpu.sync_copy(ids_hbm.at[pl.ds(base, ROWS)], ids_v)

    neg = jnp.full((LANES,), -jnp.inf, dtype=jnp.float32)
    for s in range(BATCH):
        for c in range(CVEC):
            acc_v[s, pl.ds(c * LANES, LANES)] = neg

    cnt = [jnp.zeros((LANES,), jnp.int32) for _ in range(BATCH - 1)]
    for j in range(ROWS // LANES):
        v = ids_v[pl.ds(j * LANES, LANES)]
        for s in range(1, BATCH):
            cnt[s - 1] = cnt[s - 1] + jnp.minimum(jnp.maximum(s - v, 0), 1)
    bounds = [jnp.int32(0)]
    for s in range(1, BATCH):
        v = cnt[s - 1]
        lanes = [v[i] for i in range(LANES)]
        while len(lanes) > 1:
            lanes = [lanes[2 * i] + lanes[2 * i + 1]
                     for i in range(len(lanes) // 2)]
        bounds.append(lanes[0])
    bounds.append(jnp.int32(ROWS))

    bufs = [buf0, buf1]
    sems = [sem0, sem1]
    copies = [None, None]
    copies[0] = pltpu.async_copy(
        patches_hbm.at[pl.ds(base, CHUNK)], bufs[0], sems[0])
    for k in range(NCHUNK):
        cur = k % 2
        if k + 1 < NCHUNK:
            copies[(k + 1) % 2] = pltpu.async_copy(
                patches_hbm.at[pl.ds(base + (k + 1) * CHUNK, CHUNK)],
                bufs[(k + 1) % 2], sems[(k + 1) % 2])
        copies[cur].wait()
        buf = bufs[cur]
        for s in range(BATCH):
            lo = jnp.clip(bounds[s] - k * CHUNK, 0, CHUNK)
            hi = jnp.clip(bounds[s + 1] - k * CHUNK, 0, CHUNK)

            @pl.when(hi > lo)
            def _run(s=s, lo=lo, hi=hi, buf=buf):
                def body(r, carry):
                    return tuple(
                        jnp.maximum(carry[c], buf[r, pl.ds(c * LANES, LANES)])
                        for c in range(CVEC))
                init = tuple(
                    acc_v[s, pl.ds(c * LANES, LANES)] for c in range(CVEC))
                res = lax.fori_loop(lo, hi, body, init)
                for c in range(CVEC):
                    acc_v[s, pl.ds(c * LANES, LANES)] = res[c]

    pltpu.sync_copy(acc_v, parts_hbm.at[wid])


_sc_partials = pl.kernel(
    _sc_body,
    out_type=jax.ShapeDtypeStruct((NW, BATCH, C), jnp.float32),
    mesh=plsc.VectorSubcoreMesh(core_axis_name="c", subcore_axis_name="s",
                                num_cores=NUM_CORES,
                                num_subcores=NUM_SUBCORES),
    scratch_types=[
        pltpu.VMEM((ROWS,), jnp.int32),
        pltpu.VMEM((CHUNK, C), jnp.float32),
        pltpu.VMEM((CHUNK, C), jnp.float32),
        pltpu.VMEM((BATCH, C), jnp.float32),
        pltpu.SemaphoreType.DMA,
        pltpu.SemaphoreType.DMA,
    ],
)


def _tc_merge_body(parts_ref, out_ref):
    out_ref[:] = jnp.max(parts_ref[:], axis=0)


def kernel(patches, patch_ids):
    parts = _sc_partials(patches, patch_ids)
    out = pl.pallas_call(
        _tc_merge_body,
        out_shape=jax.ShapeDtypeStruct((BATCH, C), jnp.float32),
    )(parts)
    return out


# speedup vs baseline: 2.8441x; 1.2559x over previous
import jax
import jax.numpy as jnp
from jax import lax
from jax.experimental import pallas as pl
from jax.experimental.pallas import tpu as pltpu
from jax.experimental.pallas import tpu_sc as plsc

N = 16384
C = 256
BATCH = 16

NUM_CORES = 2
NUM_SUBCORES = 16
NW = NUM_CORES * NUM_SUBCORES
ROWS = N // NW
CHUNK = 128
NCHUNK = ROWS // CHUNK
LANES = 16
CVEC = C // LANES


def _sc_body(patches_hbm, ids_hbm, parts_hbm, ids_v, buf0, buf1, acc_v,
             sem0, sem1):
    wid = lax.axis_index("s") * NUM_CORES + lax.axis_index("c")
    base = wid * ROWS

    pltpu.sync_copy(ids_hbm.at[pl.ds(base, ROWS)], ids_v)
    pltpu.async_copy(patches_hbm.at[pl.ds(base, CHUNK)], buf0, sem0)
    pltpu.async_copy(patches_hbm.at[pl.ds(base + CHUNK, CHUNK)], buf1, sem1)

    neg = jnp.full((LANES,), -jnp.inf, dtype=jnp.float32)

    def ibody(s, _):
        def icb(c, _):
            acc_v[s, pl.ds(c * LANES, LANES)] = neg
            return 0
        return lax.fori_loop(0, CVEC, icb, 0)
    lax.fori_loop(0, BATCH, ibody, 0)

    def cbody(j, cnt):
        v = ids_v[pl.ds(j * LANES, LANES)]
        return tuple(cnt[s - 1] + jnp.minimum(jnp.maximum(s - v, 0), 1)
                     for s in range(1, BATCH))
    cnt = lax.fori_loop(
        0, ROWS // LANES, cbody,
        tuple(jnp.zeros((LANES,), jnp.int32) for _ in range(BATCH - 1)))
    iota = lax.iota(jnp.int32, LANES)
    idxs = [jnp.bitwise_and(iota + k, LANES - 1) for k in (8, 4, 2, 1)]
    bounds = [jnp.int32(0)]
    for s in range(1, BATCH):
        a = cnt[s - 1]
        for idx in idxs:
            a = a + a.at[idx].get(mode="promise_in_bounds")
        bounds.append(a[0])
    bounds.append(jnp.int32(ROWS))

    def compute_chunk(k, buf):
        for s in range(BATCH):
            lo = jnp.clip(bounds[s] - k * CHUNK, 0, CHUNK)
            hi = jnp.clip(bounds[s + 1] - k * CHUNK, 0, CHUNK)

            @pl.when(hi > lo)
            def _run(s=s, lo=lo, hi=hi, buf=buf):
                def fbody(r, carry):
                    return tuple(
                        jnp.maximum(carry[c], buf[r, pl.ds(c * LANES, LANES)])
                        for c in range(CVEC))
                init = tuple(
                    acc_v[s, pl.ds(c * LANES, LANES)] for c in range(CVEC))
                res = lax.fori_loop(lo, hi, fbody, init)
                for c in range(CVEC):
                    acc_v[s, pl.ds(c * LANES, LANES)] = res[c]

    def gbody(g, _):
        k0 = 2 * g
        pltpu.make_async_copy(
            patches_hbm.at[pl.ds(base + k0 * CHUNK, CHUNK)], buf0,
            sem0).wait()
        compute_chunk(k0, buf0)

        @pl.when(k0 + 2 < NCHUNK)
        def _():
            pltpu.async_copy(
                patches_hbm.at[pl.ds(base + (k0 + 2) * CHUNK, CHUNK)],
                buf0, sem0)

        k1 = 2 * g + 1
        pltpu.make_async_copy(
            patches_hbm.at[pl.ds(base + k1 * CHUNK, CHUNK)], buf1,
            sem1).wait()
        compute_chunk(k1, buf1)

        @pl.when(k1 + 2 < NCHUNK)
        def _():
            pltpu.async_copy(
                patches_hbm.at[pl.ds(base + (k1 + 2) * CHUNK, CHUNK)],
                buf1, sem1)
        return 0
    lax.fori_loop(0, NCHUNK // 2, gbody, 0)

    pltpu.sync_copy(acc_v, parts_hbm.at[wid])


_sc_partials = pl.kernel(
    _sc_body,
    out_type=jax.ShapeDtypeStruct((NW, BATCH, C), jnp.float32),
    mesh=plsc.VectorSubcoreMesh(core_axis_name="c", subcore_axis_name="s",
                                num_cores=NUM_CORES,
                                num_subcores=NUM_SUBCORES),
    scratch_types=[
        pltpu.VMEM((ROWS,), jnp.int32),
        pltpu.VMEM((CHUNK, C), jnp.float32),
        pltpu.VMEM((CHUNK, C), jnp.float32),
        pltpu.VMEM((BATCH, C), jnp.float32),
        pltpu.SemaphoreType.DMA,
        pltpu.SemaphoreType.DMA,
    ],
)


def _tc_merge_body(parts_ref, out_ref):
    out_ref[:] = jnp.max(parts_ref[:], axis=0)


def kernel(patches, patch_ids):
    parts = _sc_partials(patches, patch_ids)
    out = pl.pallas_call(
        _tc_merge_body,
        out_shape=jax.ShapeDtypeStruct((BATCH, C), jnp.float32),
    )(parts)
    return out
